# Initial kernel scaffold; baseline (speedup 1.0000x reference)
import functools

import jax
import jax.numpy as jnp
from jax import lax
from jax.experimental import pallas as pl
from jax.experimental.pallas import tpu as pltpu
from jax.experimental.pallas import tpu_sc as plsc

ROWS = 16384 * 50
D = 64
NW = 32
BPW = ROWS // NW
CHUNK = 128
NCH = BPW // CHUNK

_mesh = plsc.VectorSubcoreMesh(core_axis_name="c", subcore_axis_name="s")


@functools.partial(
    pl.kernel,
    mesh=_mesh,
    out_type=jax.ShapeDtypeStruct((ROWS, D), jnp.float32),
    scratch_types=[
        pltpu.VMEM((NCH, CHUNK), jnp.int32),
        pltpu.VMEM((CHUNK, D), jnp.float32),
        pltpu.SemaphoreType.DMA,
    ],
)
def _sc_gather(idx_hbm, table_hbm, out_hbm, idx_v, buf, gsem):
    wid = lax.axis_index("s") * 2 + lax.axis_index("c")
    pltpu.sync_copy(idx_hbm.at[pl.ds(wid * NCH, NCH)], idx_v)
    base = wid * BPW

    def body(g, carry):
        pltpu.async_copy(table_hbm.at[idx_v.at[g]], buf, gsem).wait()
        pltpu.sync_copy(buf, out_hbm.at[pl.ds(base + g * CHUNK, CHUNK)])
        return carry

    lax.fori_loop(0, NCH, body, 0)


def kernel(x, W):
    idx2d = x.reshape(NW * NCH, CHUNK).astype(jnp.int32)
    out = _sc_gather(idx2d, W)
    return out.reshape(x.shape[0], x.shape[1], D)


# speedup vs baseline: 1.3408x; 1.3408x over previous
import functools

import jax
import jax.numpy as jnp
from jax import lax
from jax.experimental import pallas as pl
from jax.experimental.pallas import tpu as pltpu
from jax.experimental.pallas import tpu_sc as plsc

VOCAB = 1000000
T = 16384
P = 50
D = 64
NW = 32
TPW = T // NW
J = 4
NSTEP = TPW // J
RING = 4
LOOK = 2

_mesh = plsc.VectorSubcoreMesh(core_axis_name="c", subcore_axis_name="s")


@functools.partial(
    pl.kernel,
    mesh=_mesh,
    out_type=jax.ShapeDtypeStruct((P, T, 2 * D), jnp.float32),
    scratch_types=[
        pltpu.VMEM((TPW, P), jnp.int32),
        pltpu.VMEM((RING, J, P, 2 * D), jnp.float32),
        pltpu.SemaphoreType.DMA((RING,)),
        pltpu.SemaphoreType.DMA((RING,)),
    ],
    compiler_params=pltpu.CompilerParams(use_tc_tiling_on_sc=False),
)
def _sc_gather(idx_hbm, table_hbm, out_hbm, idx_v, buf, gsem, ssem):
    wid = lax.axis_index("s") * 2 + lax.axis_index("c")
    base = wid * TPW
    pltpu.sync_copy(idx_hbm.at[pl.ds(base, TPW)], idx_v)

    def gathers(step, slot):
        for j in range(J):
            pltpu.make_async_copy(
                table_hbm.at[idx_v.at[step * J + j]], buf.at[slot, j],
                gsem.at[slot],
            ).start()

    def wait_gathers(step, slot):
        for j in range(J):
            pltpu.make_async_copy(
                table_hbm.at[idx_v.at[step * J + j]], buf.at[slot, j],
                gsem.at[slot],
            ).wait()

    def stores(step, slot):
        for p in range(P):
            pltpu.make_async_copy(
                buf.at[slot, :, p, :],
                out_hbm.at[p, pl.ds(base + step * J, J)],
                ssem.at[slot],
            ).start()

    def wait_stores(step, slot):
        for p in range(P):
            pltpu.make_async_copy(
                buf.at[slot, :, p, :],
                out_hbm.at[p, pl.ds(base + step * J, J)],
                ssem.at[slot],
            ).wait()

    for b in range(LOOK):
        gathers(b, b)

    def body(t, carry):
        slot = lax.rem(t, RING)
        tgt = t + LOOK
        slot_t = lax.rem(tgt, RING)

        @pl.when(tgt < NSTEP)
        def _issue():
            @pl.when(tgt >= RING)
            def _drain():
                wait_stores(tgt - RING, slot_t)

            gathers(tgt, slot_t)

        wait_gathers(t, slot)
        stores(t, slot)
        return carry

    lax.fori_loop(0, NSTEP, body, 0)

    for i in range(LOOK):
        c = NSTEP - LOOK + i
        wait_stores(c, c % RING)


_WB = 2048


def _wformat_body(i_ref, o_ref):
    o_ref[:, 0:D] = i_ref[...].T


def _tc_wformat(wt):
    grid = (VOCAB + _WB - 1) // _WB
    return pl.pallas_call(
        _wformat_body,
        grid=(grid,),
        in_specs=[pl.BlockSpec((D, _WB), lambda k: (0, k))],
        out_specs=pl.BlockSpec((_WB, 2 * D), lambda k: (k, 0)),
        out_shape=jax.ShapeDtypeStruct((VOCAB, 2 * D), jnp.float32),
    )(wt)


_TB = 512


def _oformat_body(i_ref, o_ref):
    o_ref[...] = i_ref[0, :, 0:D].T


def _tc_oformat(gw):
    return pl.pallas_call(
        _oformat_body,
        grid=(P, T // _TB),
        in_specs=[pl.BlockSpec((1, _TB, 2 * D), lambda p, k: (p, k, 0))],
        out_specs=pl.BlockSpec((D, _TB), lambda p, k: (p, k)),
        out_shape=jax.ShapeDtypeStruct((P * D, T), jnp.float32),
    )(gw)


def kernel(x, W):
    idx = x.astype(jnp.int32)
    wt = W.T
    table = _tc_wformat(wt)
    gw = _sc_gather(idx, table)
    r2 = _tc_oformat(gw)
    return r2.reshape(P, D, T).transpose(2, 0, 1)


# speedup vs baseline: 2.4387x; 1.8189x over previous
import functools

import jax
import jax.numpy as jnp
from jax import lax
from jax.experimental import pallas as pl
from jax.experimental.pallas import tpu as pltpu
from jax.experimental.pallas import tpu_sc as plsc

VOCAB = 1000000
T = 16384
P = 50
D = 64
NW = 32
TPW = T // NW
J = 4
NSTEP = TPW // J
RING = 4
LOOK = 2

_mesh = plsc.VectorSubcoreMesh(core_axis_name="c", subcore_axis_name="s")


@functools.partial(
    pl.kernel,
    mesh=_mesh,
    out_type=jax.ShapeDtypeStruct((P, T, 2 * D), jnp.float32),
    scratch_types=[
        pltpu.VMEM((TPW, P), jnp.int32),
        pltpu.VMEM((RING, J, P, 2 * D), jnp.float32),
        pltpu.SemaphoreType.DMA((RING,)),
        pltpu.SemaphoreType.DMA((RING,)),
    ],
    compiler_params=pltpu.CompilerParams(use_tc_tiling_on_sc=False),
)
def _sc_gather(idx_hbm, table_hbm, out_hbm, idx_v, buf, gsem, ssem):
    wid = lax.axis_index("s") * 2 + lax.axis_index("c")
    base = wid * TPW
    pltpu.sync_copy(idx_hbm.at[pl.ds(base, TPW)], idx_v)

    def gathers(step, slot):
        for j in range(J):
            pltpu.make_async_copy(
                table_hbm.at[idx_v.at[step * J + j]], buf.at[slot, j],
                gsem.at[slot],
            ).start()

    def wait_gathers(step, slot):
        for j in range(J):
            pltpu.make_async_copy(
                table_hbm.at[idx_v.at[step * J + j]], buf.at[slot, j],
                gsem.at[slot],
            ).wait()

    def stores(step, slot):
        for p in range(P):
            pltpu.make_async_copy(
                buf.at[slot, :, p, :],
                out_hbm.at[p, pl.ds(base + step * J, J)],
                ssem.at[slot],
            ).start()

    def wait_stores(step, slot):
        for p in range(P):
            pltpu.make_async_copy(
                buf.at[slot, :, p, :],
                out_hbm.at[p, pl.ds(base + step * J, J)],
                ssem.at[slot],
            ).wait()

    for b in range(LOOK):
        gathers(b, b)

    def body(t, carry):
        slot = lax.rem(t, RING)
        tgt = t + LOOK
        slot_t = lax.rem(tgt, RING)

        @pl.when(tgt < NSTEP)
        def _issue():
            @pl.when(tgt >= RING)
            def _drain():
                wait_stores(tgt - RING, slot_t)

            gathers(tgt, slot_t)

        wait_gathers(t, slot)
        stores(t, slot)
        return carry

    lax.fori_loop(0, NSTEP, body, 0)

    for i in range(LOOK):
        c = NSTEP - LOOK + i
        wait_stores(c, c % RING)


_WB = 8192


def _wformat_body(i_ref, o_ref):
    o_ref[:, 0:D] = i_ref[...].T


def _tc_wformat(wt):
    grid = (VOCAB + _WB - 1) // _WB
    return pl.pallas_call(
        _wformat_body,
        grid=(grid,),
        in_specs=[pl.BlockSpec((D, _WB), lambda k: (0, k))],
        out_specs=pl.BlockSpec((_WB, 2 * D), lambda k: (k, 0)),
        out_shape=jax.ShapeDtypeStruct((VOCAB, 2 * D), jnp.float32),
    )(wt)


_TB = 2048


def _oformat_body(i_ref, o_ref):
    o_ref[...] = i_ref[0].T[0:D]


def _tc_oformat(gw):
    return pl.pallas_call(
        _oformat_body,
        grid=(P, T // _TB),
        in_specs=[pl.BlockSpec((1, _TB, 2 * D), lambda p, k: (p, k, 0))],
        out_specs=pl.BlockSpec((D, _TB), lambda p, k: (p, k)),
        out_shape=jax.ShapeDtypeStruct((P * D, T), jnp.float32),
    )(gw)


def kernel(x, W):
    idx = x.astype(jnp.int32)
    wt = W.T
    table = _tc_wformat(wt)
    gw = _sc_gather(idx, table)
    r2 = _tc_oformat(gw)
    return r2.reshape(P, D, T).transpose(2, 0, 1)
